# Initial kernel scaffold; baseline (speedup 1.0000x reference)
import jax
import jax.numpy as jnp
from jax.experimental import pallas as pl


def kernel(stm_indices, nstm_indices, table, input_bias, W_hidden, b_hidden):
    raise NotImplementedError("write your pallas kernel here")


# speedup vs baseline: 17.1383x; 17.1383x over previous
import functools

import jax
import jax.numpy as jnp
from jax import lax
from jax.experimental import pallas as pl
from jax.experimental.pallas import tpu as pltpu
from jax.experimental.pallas import tpu_sc as plsc

N_F = 768
NF_PAD = 896
B = 16384
T = 32
L1 = 1024
NC, NS, L = 2, 16, 16
NW = NC * NS
ROWS_PER_W = B // NW
CHUNK = 64
VECS_PER_CHUNK = CHUNK * T // L


def _sc_counts_body(stm_hbm, nstm_hbm, cs_hbm, cn_hbm, idx_v, cnt_v):
    wid = lax.axis_index("s") * NC + lax.axis_index("c")
    base_row = wid * ROWS_PER_W
    lane = lax.iota(jnp.int32, L)
    zeros16 = jnp.zeros((L,), jnp.float32)
    plus1 = jnp.full((L,), 1.0, jnp.float32)
    minus1 = jnp.full((L,), -1.0, jnp.float32)

    def zero_body(i, c):
        cnt_v[pl.ds(i * L, L)] = zeros16
        return c
    lax.fori_loop(0, CHUNK * NF_PAD // L, zero_body, 0)

    def scatter_pass(val16):
        def scat(j, c):
            e = j * L + lane
            idx16 = idx_v[pl.ds(j * L, L)]
            off = (e >> 5) * NF_PAD + idx16
            plsc.addupdate_scatter(cnt_v, [off], val16)
            return c
        lax.fori_loop(0, VECS_PER_CHUNK, scat, 0)

    for src, dst in ((stm_hbm, cs_hbm), (nstm_hbm, cn_hbm)):
        def chunk_body(c, _, src=src, dst=dst):
            row0 = base_row + c * CHUNK
            pltpu.sync_copy(src.at[pl.ds(row0 * T, CHUNK * T)], idx_v)
            scatter_pass(plus1)
            pltpu.sync_copy(cnt_v, dst.at[pl.ds(row0 * NF_PAD, CHUNK * NF_PAD)])
            scatter_pass(minus1)
            return 0
        lax.fori_loop(0, ROWS_PER_W // CHUNK, chunk_body, 0)


@functools.cache
def _sc_counts():
    return pl.kernel(
        _sc_counts_body,
        out_type=(
            jax.ShapeDtypeStruct((B * NF_PAD,), jnp.float32),
            jax.ShapeDtypeStruct((B * NF_PAD,), jnp.float32),
        ),
        mesh=plsc.VectorSubcoreMesh(core_axis_name="c", subcore_axis_name="s"),
        scratch_types=[
            pltpu.VMEM((CHUNK * T,), jnp.int32),
            pltpu.VMEM((CHUNK * NF_PAD,), jnp.float32),
        ],
        compiler_params=pltpu.CompilerParams(needs_layout_passes=False),
    )


BB = 512


def _tc_body(cs_ref, cn_ref, tab_ref, bias_ref, w_ref, bh_ref, out_ref):
    tab = tab_ref[...]
    cs = cs_ref[...]
    cn = cn_ref[...]
    bias = bias_ref[...]
    emb_s = jnp.dot(cs.astype(jnp.bfloat16), tab,
                    preferred_element_type=jnp.float32)
    emb_s = jnp.clip(emb_s + bias, 0.0, 1.0)
    emb_n = jnp.dot(cn.astype(jnp.bfloat16), tab,
                    preferred_element_type=jnp.float32)
    emb_n = jnp.clip(emb_n + bias, 0.0, 1.0)
    w = w_ref[...]
    hs = lax.dot_general(emb_s, w[:, :L1], (((1,), (1,)), ((), ())),
                         preferred_element_type=jnp.float32)
    hn = lax.dot_general(emb_n, w[:, L1:], (((1,), (1,)), ((), ())),
                         preferred_element_type=jnp.float32)
    heads = hs + hn + bh_ref[...]
    n_active = (jnp.float32(T) - cs[:, N_F:N_F + 1]).astype(jnp.int32)
    bucket = jnp.clip((n_active - 2) >> 2, 0, 7)
    hsel = jnp.where(
        lax.broadcasted_iota(jnp.int32, (BB, 8), 1) == bucket, heads, 0.0)
    out_ref[...] = jnp.sum(hsel, axis=1, keepdims=True)


_tc_forward = pl.pallas_call(
    _tc_body,
    grid=(B // BB,),
    in_specs=[
        pl.BlockSpec((BB, NF_PAD), lambda i: (i, 0)),
        pl.BlockSpec((BB, NF_PAD), lambda i: (i, 0)),
        pl.BlockSpec((NF_PAD, L1), lambda i: (0, 0)),
        pl.BlockSpec((1, L1), lambda i: (0, 0)),
        pl.BlockSpec((8, 2 * L1), lambda i: (0, 0)),
        pl.BlockSpec((1, 8), lambda i: (0, 0)),
    ],
    out_specs=pl.BlockSpec((BB, 1), lambda i: (i, 0)),
    out_shape=jax.ShapeDtypeStruct((B, 1), jnp.float32),
)


def kernel(stm_indices, nstm_indices, table, input_bias, W_hidden, b_hidden):
    stm_flat = stm_indices.reshape(-1).astype(jnp.int32)
    nstm_flat = nstm_indices.reshape(-1).astype(jnp.int32)
    cs_flat, cn_flat = _sc_counts()(stm_flat, nstm_flat)
    cs = cs_flat.reshape(B, NF_PAD)
    cn = cn_flat.reshape(B, NF_PAD)
    tab = jnp.zeros((NF_PAD, L1), jnp.float32).at[:N_F + 1].set(table)
    tab = tab.at[N_F].set(0.0).astype(jnp.bfloat16)
    return _tc_forward(cs, cn, tab, input_bias.reshape(1, L1),
                       W_hidden, b_hidden.reshape(1, 8))
